# baseline (speedup 1.0000x reference)
import functools

import jax
import jax.numpy as jnp
from jax import lax
from jax.experimental import pallas as pl
from jax.experimental.pallas import tpu as pltpu
from jax.experimental.pallas import tpu_sc as plsc

VOCAB_SIZE = 1000000
WIDTH = 64
BATCH = 16384
HIST = 50

NC = 2
NS = 16
NW = NC * NS

G = 128
BT_PER_W = BATCH // (NW * G)
NBLK = HIST * BT_PER_W
OUT_ROWS = HIST * (WIDTH // 8) * (BATCH // G)


def _embed_kernel(ids_hbm, table_hbm, out_hbm,
                  idx_v, gb0, gb1, tb0, tb1, gs0, gs1, ws0, ws1):
    wid = lax.axis_index("s") * NC + lax.axis_index("c")

    pltpu.sync_copy(ids_hbm.at[wid], idx_v)

    lane = lax.iota(jnp.int32, 16)

    def fire_gather(gb, gs, n):
        pltpu.async_copy(table_hbm.at[idx_v.at[n]], gb, gs)

    def drain_gather(gb, gs):
        pltpu.make_async_copy(table_hbm.at[pl.ds(0, G)], gb, gs).wait()

    def transpose(gb, tb):
        def col(d, carry):
            dt = lax.shift_right_logical(d, 3)
            base = lax.shift_left(lax.bitwise_and(d, 7), 7)
            cidx = jnp.full((16,), 0, jnp.int32) + d
            for g in range(8):
                vec = plsc.load_gather(gb, [lane + (16 * g), cidx])
                tb[dt, pl.ds(base + 16 * g, 16)] = vec
            return carry
        lax.fori_loop(0, WIDTH, col, 0, unroll=2)

    def fire_writes(tb, ws, n):
        h = lax.shift_right_logical(n, 2)
        bt = lax.bitwise_and(n, 3)
        r0 = h * 1024 + wid * BT_PER_W + bt
        for dt in range(8):
            pltpu.async_copy(tb.at[dt], out_hbm.at[r0 + dt * 128], ws)

    def drain_writes(tb, ws):
        pltpu.make_async_copy(tb, out_hbm.at[pl.ds(0, 8)], ws).wait()

    fire_gather(gb0, gs0, 0)
    fire_gather(gb1, gs1, 1)

    def half(g, n, gb, gs, tb, ws):
        drain_gather(gb, gs)

        @pl.when(g > 0)
        def _():
            drain_writes(tb, ws)

        transpose(gb, tb)
        fire_writes(tb, ws, n)
        fire_gather(gb, gs, n + 2)

    def body(g, carry):
        half(g, 2 * g, gb0, gs0, tb0, ws0)
        half(g, 2 * g + 1, gb1, gs1, tb1, ws1)
        return carry

    lax.fori_loop(0, NBLK // 2 - 1, body, 0)

    for (n, gb, gs, tb, ws) in ((NBLK - 2, gb0, gs0, tb0, ws0),
                                (NBLK - 1, gb1, gs1, tb1, ws1)):
        drain_gather(gb, gs)
        drain_writes(tb, ws)
        transpose(gb, tb)
        fire_writes(tb, ws, n)
    drain_writes(tb0, ws0)
    drain_writes(tb1, ws1)


@jax.jit
def _embed(ids3, table):
    mesh = plsc.VectorSubcoreMesh(core_axis_name="c", subcore_axis_name="s")
    out = pl.kernel(
        _embed_kernel,
        out_type=jax.ShapeDtypeStruct((OUT_ROWS, 1024), jnp.float32),
        mesh=mesh,
        scratch_types=[
            pltpu.VMEM((NBLK, G), jnp.int32),
            pltpu.VMEM((G, WIDTH), jnp.float32),
            pltpu.VMEM((G, WIDTH), jnp.float32),
            pltpu.VMEM((8, 1024), jnp.float32),
            pltpu.VMEM((8, 1024), jnp.float32),
            pltpu.SemaphoreType.DMA,
            pltpu.SemaphoreType.DMA,
            pltpu.SemaphoreType.DMA,
            pltpu.SemaphoreType.DMA,
        ],
        compiler_params=pltpu.CompilerParams(
            use_tc_tiling_on_sc=False, needs_layout_passes=False
        ),
    )(ids3, table)
    return (
        out.reshape(HIST, 8, BATCH // G, 8, G)
        .transpose(2, 4, 0, 1, 3)
        .reshape(BATCH, HIST, WIDTH)
    )


def kernel(input_ids, table):
    ids3 = (
        input_ids.astype(jnp.int32)
        .reshape(NW, BT_PER_W, G, HIST)
        .transpose(0, 3, 1, 2)
        .reshape(NW, NBLK, G)
    )
    return _embed(ids3, table)


# speedup vs baseline: 1.1397x; 1.1397x over previous
import functools

import jax
import jax.numpy as jnp
from jax import lax
from jax.experimental import pallas as pl
from jax.experimental.pallas import tpu as pltpu
from jax.experimental.pallas import tpu_sc as plsc

VOCAB_SIZE = 1000000
WIDTH = 64
BATCH = 16384
HIST = 50

NC = 2
NS = 16
NW = NC * NS

G = 128
BT_PER_W = BATCH // (NW * G)
NBLK = HIST * BT_PER_W
NSETS = 5
OUT_ROWS = HIST * (WIDTH // 8) * (BATCH // G)


def _embed_kernel(ids_hbm, table_hbm, out_hbm, idx_v, *bufs):
    gbs = bufs[0:NSETS]
    tbs = bufs[NSETS:2 * NSETS]
    gss = bufs[2 * NSETS:3 * NSETS]
    wss = bufs[3 * NSETS:4 * NSETS]

    wid = lax.axis_index("s") * NC + lax.axis_index("c")

    pltpu.sync_copy(ids_hbm.at[wid], idx_v)

    lane = lax.iota(jnp.int32, 16)
    lane16 = [lane + 16 * g for g in range(8)]

    def fire_gather(gb, gs, n):
        pltpu.async_copy(table_hbm.at[idx_v.at[2 * n]], gb.at[pl.ds(0, 64)], gs)
        pltpu.async_copy(table_hbm.at[idx_v.at[2 * n + 1]], gb.at[pl.ds(64, 64)], gs)

    def drain_gather(gb, gs):
        pltpu.make_async_copy(table_hbm.at[pl.ds(0, G)], gb, gs).wait()

    def transpose(gb, tb):
        def col2(i, carry):
            for dd in range(2):
                d = 2 * i + dd
                dt = lax.shift_right_logical(d, 3)
                base = lax.shift_left(lax.bitwise_and(d, 7), 7)
                cidx = jnp.full((16,), d, jnp.int32)
                vecs = [plsc.load_gather(gb, [lane16[g], cidx]) for g in range(8)]
                for g in range(8):
                    tb[dt, pl.ds(base + 16 * g, 16)] = vecs[g]
            return carry
        lax.fori_loop(0, WIDTH // 2, col2, 0)

    def fire_writes(tb, ws, n):
        h = lax.shift_right_logical(n, 2)
        bt = lax.bitwise_and(n, 3)
        r0 = h * 1024 + wid * BT_PER_W + bt
        for dt in range(8):
            pltpu.async_copy(tb.at[dt], out_hbm.at[r0 + dt * 128], ws)

    def drain_writes(tb, ws):
        pltpu.make_async_copy(tb, out_hbm.at[pl.ds(0, 8)], ws).wait()

    for s in range(NSETS):
        fire_gather(gbs[s], gss[s], s)

    def half(g, n, gb, gs, tb, ws):
        drain_gather(gb, gs)

        @pl.when(g > 0)
        def _():
            drain_writes(tb, ws)

        transpose(gb, tb)
        fire_writes(tb, ws, n)
        fire_gather(gb, gs, n + NSETS)

    def body(g, carry):
        for s in range(NSETS):
            half(g, NSETS * g + s, gbs[s], gss[s], tbs[s], wss[s])
        return carry

    lax.fori_loop(0, NBLK // NSETS - 1, body, 0)

    for s in range(NSETS):
        n = NBLK - NSETS + s
        drain_gather(gbs[s], gss[s])
        drain_writes(tbs[s], wss[s])
        transpose(gbs[s], tbs[s])
        fire_writes(tbs[s], wss[s], n)
    for s in range(NSETS):
        drain_writes(tbs[s], wss[s])


@jax.jit
def _embed(ids3, table):
    mesh = plsc.VectorSubcoreMesh(core_axis_name="c", subcore_axis_name="s")
    out = pl.kernel(
        _embed_kernel,
        out_type=jax.ShapeDtypeStruct((OUT_ROWS, 1024), jnp.float32),
        mesh=mesh,
        scratch_types=(
            [pltpu.VMEM((2 * NBLK, 64), jnp.int32)]
            + [pltpu.VMEM((G, WIDTH), jnp.float32) for _ in range(NSETS)]
            + [pltpu.VMEM((8, 1024), jnp.float32) for _ in range(NSETS)]
            + [pltpu.SemaphoreType.DMA for _ in range(2 * NSETS)]
        ),
        compiler_params=pltpu.CompilerParams(
            use_tc_tiling_on_sc=False, needs_layout_passes=False
        ),
    )(ids3, table)
    return (
        out.reshape(HIST, 8, BATCH // G, 8, G)
        .transpose(2, 4, 0, 1, 3)
        .reshape(BATCH, HIST, WIDTH)
    )


def kernel(input_ids, table):
    ids3 = (
        input_ids.astype(jnp.int32)
        .reshape(NW, BT_PER_W, 2, 64, HIST)
        .transpose(0, 4, 1, 2, 3)
        .reshape(NW, 2 * NBLK, 64)
    )
    return _embed(ids3, table)


# speedup vs baseline: 1.4918x; 1.3089x over previous
import functools

import jax
import jax.numpy as jnp
from jax import lax
from jax.experimental import pallas as pl
from jax.experimental.pallas import tpu as pltpu
from jax.experimental.pallas import tpu_sc as plsc

VOCAB_SIZE = 1000000
WIDTH = 64
BATCH = 16384
HIST = 50

NC = 2
NS = 16
NW = NC * NS

G = 128
BT_PER_W = BATCH // (NW * G)
NBLK = HIST * BT_PER_W
NSETS = 5
OUT_ROWS = HIST * (WIDTH // 8) * (BATCH // G)


def _embed_kernel(ids_hbm, table_hbm, out_hbm, idx_v, *bufs):
    gbs = bufs[0:NSETS]
    tbs = bufs[NSETS:2 * NSETS]
    gss = bufs[2 * NSETS:3 * NSETS]
    wss = bufs[3 * NSETS:4 * NSETS]

    wid = lax.axis_index("s") * NC + lax.axis_index("c")

    pltpu.sync_copy(ids_hbm.at[wid], idx_v)

    lane = lax.iota(jnp.int32, 16)
    lane16 = [lane + 16 * g for g in range(8)]

    def fire_gather(gb, gs, n):
        pltpu.async_copy(table_hbm.at[idx_v.at[2 * n]], gb.at[pl.ds(0, 64)], gs)
        pltpu.async_copy(table_hbm.at[idx_v.at[2 * n + 1]], gb.at[pl.ds(64, 64)], gs)

    def drain_gather(gb, gs):
        pltpu.make_async_copy(table_hbm.at[pl.ds(0, G)], gb, gs).wait()

    def transpose(gb, tb):
        def gbody(g, carry):
            jvec = lane + g * 16
            for v in range(16):
                rot = lax.bitwise_and(lane + v, 15)
                for c in range(4):
                    dvec = rot + (16 * c)
                    vec = plsc.load_gather(gb, [jvec, dvec])
                    dt = lax.shift_right_logical(dvec, 3)
                    col = lax.shift_left(lax.bitwise_and(dvec, 7), 7) + jvec
                    plsc.store_scatter(tb, [dt, col], vec)
            return carry
        lax.fori_loop(0, G // 16, gbody, 0)

    def fire_writes(tb, ws, n):
        h = lax.shift_right_logical(n, 2)
        bt = lax.bitwise_and(n, 3)
        r0 = h * 1024 + wid * BT_PER_W + bt
        for dt in range(8):
            pltpu.async_copy(tb.at[dt], out_hbm.at[r0 + dt * 128], ws)

    def drain_writes(tb, ws):
        pltpu.make_async_copy(tb, out_hbm.at[pl.ds(0, 8)], ws).wait()

    for s in range(NSETS):
        fire_gather(gbs[s], gss[s], s)

    def half(g, n, gb, gs, tb, ws):
        drain_gather(gb, gs)

        @pl.when(g > 0)
        def _():
            drain_writes(tb, ws)

        transpose(gb, tb)
        fire_writes(tb, ws, n)
        fire_gather(gb, gs, n + NSETS)

    def body(g, carry):
        for s in range(NSETS):
            half(g, NSETS * g + s, gbs[s], gss[s], tbs[s], wss[s])
        return carry

    lax.fori_loop(0, NBLK // NSETS - 1, body, 0)

    for s in range(NSETS):
        n = NBLK - NSETS + s
        drain_gather(gbs[s], gss[s])
        drain_writes(tbs[s], wss[s])
        transpose(gbs[s], tbs[s])
        fire_writes(tbs[s], wss[s], n)
    for s in range(NSETS):
        drain_writes(tbs[s], wss[s])


@jax.jit
def _embed(ids3, table):
    mesh = plsc.VectorSubcoreMesh(core_axis_name="c", subcore_axis_name="s")
    out = pl.kernel(
        _embed_kernel,
        out_type=jax.ShapeDtypeStruct((OUT_ROWS, 1024), jnp.float32),
        mesh=mesh,
        scratch_types=(
            [pltpu.VMEM((2 * NBLK, 64), jnp.int32)]
            + [pltpu.VMEM((G, WIDTH), jnp.float32) for _ in range(NSETS)]
            + [pltpu.VMEM((8, 1024), jnp.float32) for _ in range(NSETS)]
            + [pltpu.SemaphoreType.DMA for _ in range(2 * NSETS)]
        ),
        compiler_params=pltpu.CompilerParams(
            use_tc_tiling_on_sc=False, needs_layout_passes=False
        ),
    )(ids3, table)
    return (
        out.reshape(HIST, 8, BATCH // G, 8, G)
        .transpose(2, 4, 0, 1, 3)
        .reshape(BATCH, HIST, WIDTH)
    )


def kernel(input_ids, table):
    ids3 = (
        input_ids.astype(jnp.int32)
        .reshape(NW, BT_PER_W, 2, 64, HIST)
        .transpose(0, 4, 1, 2, 3)
        .reshape(NW, 2 * NBLK, 64)
    )
    return _embed(ids3, table)


# speedup vs baseline: 1.7269x; 1.1576x over previous
import functools

import jax
import jax.numpy as jnp
from jax import lax
from jax.experimental import pallas as pl
from jax.experimental.pallas import tpu as pltpu
from jax.experimental.pallas import tpu_sc as plsc

VOCAB_SIZE = 1000000
WIDTH = 64
BATCH = 16384
HIST = 50

NC = 2
NS = 16
NW = NC * NS

G = 128
BT_PER_W = BATCH // (NW * G)
NBLK = HIST * BT_PER_W
NSETS = 5
OUT_ROWS = HIST * (WIDTH // 8) * (BATCH // G)


def _embed_kernel(ids_hbm, table_hbm, out_hbm, idx_v, *bufs):
    gbs = bufs[0:NSETS]
    tbs = bufs[NSETS:2 * NSETS]
    gss = bufs[2 * NSETS:3 * NSETS]
    wss = bufs[3 * NSETS:4 * NSETS]

    wid = lax.axis_index("s") * NC + lax.axis_index("c")

    pltpu.sync_copy(ids_hbm.at[wid], idx_v)

    lane = lax.iota(jnp.int32, 16)

    def fire_gather(gb, gs, n):
        pltpu.async_copy(table_hbm.at[idx_v.at[2 * n]], gb.at[pl.ds(0, 64)], gs)
        pltpu.async_copy(table_hbm.at[idx_v.at[2 * n + 1]], gb.at[pl.ds(64, 64)], gs)

    def drain_gather(gb, gs):
        pltpu.make_async_copy(table_hbm.at[pl.ds(0, G)], gb, gs).wait()

    zero16 = lane - lane
    l64 = lax.shift_left(lane, 6)
    load_tab = []
    store_tab = []
    for v in range(16):
        r = lax.bitwise_and(lane + v, 15)
        load_tab.append(l64 + r)
        store_tab.append(
            lax.shift_left(lax.shift_right_logical(r, 3), 10)
            + lax.shift_left(lax.bitwise_and(r, 7), 7)
            + lane
        )

    def transpose(gb, tb):
        def gbody(g, carry):
            g1024 = g * 1024
            g16 = g * 16
            for v in range(16):
                for c in range(4):
                    vec = plsc.load_gather(gb, [zero16, load_tab[v] + (g1024 + 16 * c)])
                    plsc.store_scatter(
                        tb, [zero16, store_tab[v] + (g16 + 2048 * c)], vec
                    )
            return carry
        lax.fori_loop(0, G // 16, gbody, 0)

    def fire_writes(tb, ws, n):
        h = lax.shift_right_logical(n, 2)
        bt = lax.bitwise_and(n, 3)
        r0 = h * 1024 + wid * BT_PER_W + bt
        for dt in range(8):
            pltpu.async_copy(tb.at[dt], out_hbm.at[r0 + dt * 128], ws)

    def drain_writes(tb, ws):
        pltpu.make_async_copy(tb, out_hbm.at[pl.ds(0, 8)], ws).wait()

    for s in range(NSETS):
        fire_gather(gbs[s], gss[s], s)

    def half(g, n, gb, gs, tb, ws):
        drain_gather(gb, gs)

        @pl.when(g > 0)
        def _():
            drain_writes(tb, ws)

        transpose(gb, tb)
        fire_writes(tb, ws, n)
        fire_gather(gb, gs, n + NSETS)

    def body(g, carry):
        for s in range(NSETS):
            half(g, NSETS * g + s, gbs[s], gss[s], tbs[s], wss[s])
        return carry

    lax.fori_loop(0, NBLK // NSETS - 1, body, 0)

    for s in range(NSETS):
        n = NBLK - NSETS + s
        drain_gather(gbs[s], gss[s])
        drain_writes(tbs[s], wss[s])
        transpose(gbs[s], tbs[s])
        fire_writes(tbs[s], wss[s], n)
    for s in range(NSETS):
        drain_writes(tbs[s], wss[s])


@jax.jit
def _embed(ids3, table):
    mesh = plsc.VectorSubcoreMesh(core_axis_name="c", subcore_axis_name="s")
    out = pl.kernel(
        _embed_kernel,
        out_type=jax.ShapeDtypeStruct((OUT_ROWS, 1024), jnp.float32),
        mesh=mesh,
        scratch_types=(
            [pltpu.VMEM((2 * NBLK, 64), jnp.int32)]
            + [pltpu.VMEM((G, WIDTH), jnp.float32) for _ in range(NSETS)]
            + [pltpu.VMEM((8, 1024), jnp.float32) for _ in range(NSETS)]
            + [pltpu.SemaphoreType.DMA for _ in range(2 * NSETS)]
        ),
        compiler_params=pltpu.CompilerParams(
            use_tc_tiling_on_sc=False, needs_layout_passes=False
        ),
    )(ids3, table)
    return (
        out.reshape(HIST, 8, BATCH // G, 8, G)
        .transpose(2, 4, 0, 1, 3)
        .reshape(BATCH, HIST, WIDTH)
    )


def kernel(input_ids, table):
    ids3 = (
        input_ids.astype(jnp.int32)
        .reshape(NW, BT_PER_W, 2, 64, HIST)
        .transpose(0, 4, 1, 2, 3)
        .reshape(NW, 2 * NBLK, 64)
    )
    return _embed(ids3, table)


# speedup vs baseline: 2.4069x; 1.3938x over previous
import functools

import jax
import jax.numpy as jnp
from jax import lax
from jax.experimental import pallas as pl
from jax.experimental.pallas import tpu as pltpu
from jax.experimental.pallas import tpu_sc as plsc

VOCAB_SIZE = 1000000
WIDTH = 64
BATCH = 16384
HIST = 50

NC = 2
NS = 16
NW = NC * NS

G = 128
BT_PER_W = BATCH // (NW * G)
NBLK = HIST * BT_PER_W
NSETS = 5
OUT_ROWS = HIST * (WIDTH // 8) * (BATCH // G)


def _embed_kernel(ids_hbm, table_hbm, out_hbm, idx_v, *bufs):
    gbs = bufs[0:NSETS]
    tbs = bufs[NSETS:2 * NSETS]
    gss = bufs[2 * NSETS:3 * NSETS]
    wss = bufs[3 * NSETS:4 * NSETS]

    wid = lax.axis_index("s") * NC + lax.axis_index("c")

    pltpu.sync_copy(ids_hbm.at[wid], idx_v)

    lane = lax.iota(jnp.int32, 16)

    def fire_gather(gb, gs, n):
        pltpu.async_copy(table_hbm.at[idx_v.at[2 * n]], gb.at[pl.ds(0, 64)], gs)
        pltpu.async_copy(table_hbm.at[idx_v.at[2 * n + 1]], gb.at[pl.ds(64, 64)], gs)

    def drain_gather(gb, gs):
        pltpu.make_async_copy(table_hbm.at[pl.ds(0, G)], gb, gs).wait()

    zero16 = lane - lane
    l64 = lax.shift_left(lane, 6)
    load_tab = []
    store_tab = []
    for v in range(16):
        r = lax.bitwise_and(lane + v, 15)
        load_tab.append(l64 + r)
        store_tab.append(
            lax.shift_left(lax.shift_right_logical(r, 3), 10)
            + lax.shift_left(lax.bitwise_and(r, 7), 7)
            + lane
        )

    def transpose(gb, tb):
        def gbody(g, carry):
            g1024 = g * 1024
            g16 = g * 16
            for v2 in range(8):
                vecs = [
                    plsc.load_gather(
                        gb, [zero16, load_tab[2 * v2 + vv] + (g1024 + 16 * c)]
                    )
                    for vv in range(2)
                    for c in range(4)
                ]
                i = 0
                for vv in range(2):
                    for c in range(4):
                        plsc.store_scatter(
                            tb,
                            [zero16, store_tab[2 * v2 + vv] + (g16 + 2048 * c)],
                            vecs[i],
                        )
                        i += 1
            return carry
        lax.fori_loop(0, G // 16, gbody, 0)

    def fire_writes(tb, ws, n):
        h = lax.shift_right_logical(n, 2)
        bt = lax.bitwise_and(n, 3)
        r0 = h * 1024 + wid * BT_PER_W + bt
        for dt in range(8):
            pltpu.async_copy(tb.at[dt], out_hbm.at[r0 + dt * 128], ws)

    def drain_writes(tb, ws):
        pltpu.make_async_copy(tb, out_hbm.at[pl.ds(0, 8)], ws).wait()

    for s in range(NSETS):
        fire_gather(gbs[s], gss[s], s)

    def half(g, n, gb, gs, tb, ws):
        drain_gather(gb, gs)

        @pl.when(g > 0)
        def _():
            drain_writes(tb, ws)

        transpose(gb, tb)
        fire_writes(tb, ws, n)
        fire_gather(gb, gs, n + NSETS)

    def body(g, carry):
        for s in range(NSETS):
            half(g, NSETS * g + s, gbs[s], gss[s], tbs[s], wss[s])
        return carry

    lax.fori_loop(0, NBLK // NSETS - 1, body, 0)

    for s in range(NSETS):
        n = NBLK - NSETS + s
        drain_gather(gbs[s], gss[s])
        drain_writes(tbs[s], wss[s])
        transpose(gbs[s], tbs[s])
        fire_writes(tbs[s], wss[s], n)
    for s in range(NSETS):
        drain_writes(tbs[s], wss[s])


@jax.jit
def _embed(ids3, table):
    mesh = plsc.VectorSubcoreMesh(core_axis_name="c", subcore_axis_name="s")
    out = pl.kernel(
        _embed_kernel,
        out_type=jax.ShapeDtypeStruct((OUT_ROWS, 1024), jnp.float32),
        mesh=mesh,
        scratch_types=(
            [pltpu.VMEM((2 * NBLK, 64), jnp.int32)]
            + [pltpu.VMEM((G, WIDTH), jnp.float32) for _ in range(NSETS)]
            + [pltpu.VMEM((8, 1024), jnp.float32) for _ in range(NSETS)]
            + [pltpu.SemaphoreType.DMA for _ in range(2 * NSETS)]
        ),
        compiler_params=pltpu.CompilerParams(
            use_tc_tiling_on_sc=False, needs_layout_passes=False
        ),
    )(ids3, table)
    return (
        out.reshape(HIST, 8, BATCH // G, 8, G)
        .transpose(2, 4, 0, 1, 3)
        .reshape(BATCH, HIST, WIDTH)
    )


def kernel(input_ids, table):
    ids3 = (
        input_ids.astype(jnp.int32)
        .reshape(NW, BT_PER_W, 2, 64, HIST)
        .transpose(0, 4, 1, 2, 3)
        .reshape(NW, 2 * NBLK, 64)
    )
    return _embed(ids3, table)
